# baseline (speedup 1.0000x reference)
import functools

import jax
import jax.numpy as jnp
from jax import lax
from jax.experimental import pallas as pl
from jax.experimental.pallas import tpu as pltpu
from jax.experimental.pallas import tpu_sc as plsc

V = 1_000_000
D = 64
B = 16384
NC = 2
NS = 16
NW = NC * NS
BPW = B // NW
LANES = 128
NTC = -(-V // LANES)


def _transpose_block(in_ref, out_ref):
    x = in_ref[...]
    row = jax.lax.broadcasted_iota(jnp.int32, (D, LANES), 0)
    col = jax.lax.broadcasted_iota(jnp.int32, (D, LANES), 1)
    eye = (row == col).astype(jnp.float32)
    z = jax.lax.dot_general(
        x, eye[:, :D], (((0,), (0,)), ((), ())),
        preferred_element_type=jnp.float32,
        precision=jax.lax.Precision.HIGHEST,
    )
    p_even = (col == 2 * row).astype(jnp.float32)
    p_odd = (col == 2 * row + 1).astype(jnp.float32)
    y_left = jax.lax.dot_general(
        p_even, z, (((1,), (0,)), ((), ())),
        preferred_element_type=jnp.float32,
        precision=jax.lax.Precision.HIGHEST,
    )
    y_right = jax.lax.dot_general(
        p_odd, z, (((1,), (0,)), ((), ())),
        preferred_element_type=jnp.float32,
        precision=jax.lax.Precision.HIGHEST,
    )
    y = jnp.concatenate([y_left, y_right], axis=1)
    out_ref[...] = y.reshape(D * LANES)


_detile = pl.pallas_call(
    _transpose_block,
    grid=(NTC,),
    in_specs=[pl.BlockSpec((D, LANES), lambda j: (0, j))],
    out_specs=pl.BlockSpec((D * LANES,), lambda j: (j,)),
    out_shape=jax.ShapeDtypeStruct((V * D,), jnp.float32),
)

_mesh = plsc.VectorSubcoreMesh(core_axis_name="c", subcore_axis_name="s")


@functools.partial(
    pl.kernel,
    mesh=_mesh,
    out_type=jax.ShapeDtypeStruct((B, D), jnp.float32),
    scratch_types=[
        pltpu.VMEM((BPW,), jnp.int32),
        pltpu.VMEM((BPW, D), jnp.float32),
        pltpu.SemaphoreType.DMA,
    ],
    compiler_params=pltpu.CompilerParams(use_tc_tiling_on_sc=False),
)
def _gather_kernel(idx_hbm, tbl_hbm, out_hbm, idx_v, rows_v, sem):
    wid = lax.axis_index("s") * NC + lax.axis_index("c")
    base = wid * BPW
    pltpu.sync_copy(idx_hbm.at[pl.ds(base, BPW)], idx_v)
    pltpu.async_copy(tbl_hbm.at[idx_v], rows_v, sem).wait()
    pltpu.sync_copy(rows_v, out_hbm.at[pl.ds(base, BPW)])


def kernel(all_answers, embedding_table):
    flat = _detile(embedding_table.T)
    return _gather_kernel(all_answers, flat.reshape(V, D))


# speedup vs baseline: 6.4420x; 6.4420x over previous
import functools

import jax
import jax.numpy as jnp
from jax import lax
from jax.experimental import pallas as pl
from jax.experimental.pallas import tpu as pltpu
from jax.experimental.pallas import tpu_sc as plsc

V = 1_000_000
D = 64
B = 16384
NC = 2
NS = 16
NW = NC * NS
BPW = B // NW
LANES = 128
NTC = -(-V // LANES)


C = 1024
NB = -(-V // C)


def _transpose_block(in_ref, out_ref):
    z = in_ref[...].T
    zp = jnp.concatenate([z, jnp.zeros_like(z)], 1)
    s = zp.reshape(C // 2, 2, LANES)
    y = s[:, 0, :] + jnp.roll(s[:, 1, :], D, axis=1)
    out_ref[...] = y.reshape(C * D)


_detile = pl.pallas_call(
    _transpose_block,
    grid=(NB,),
    in_specs=[pl.BlockSpec((D, C), lambda j: (0, j))],
    out_specs=pl.BlockSpec((C * D,), lambda j: (j,)),
    out_shape=jax.ShapeDtypeStruct((V * D,), jnp.float32),
)

_mesh = plsc.VectorSubcoreMesh(core_axis_name="c", subcore_axis_name="s")


@functools.partial(
    pl.kernel,
    mesh=_mesh,
    out_type=jax.ShapeDtypeStruct((B, D), jnp.float32),
    scratch_types=[
        pltpu.VMEM((BPW,), jnp.int32),
        pltpu.VMEM((BPW, D), jnp.float32),
        pltpu.SemaphoreType.DMA,
    ],
    compiler_params=pltpu.CompilerParams(use_tc_tiling_on_sc=False),
)
def _gather_kernel(idx_hbm, tbl_hbm, out_hbm, idx_v, rows_v, sem):
    wid = lax.axis_index("s") * NC + lax.axis_index("c")
    base = wid * BPW
    pltpu.sync_copy(idx_hbm.at[pl.ds(base, BPW)], idx_v)
    pltpu.async_copy(tbl_hbm.at[idx_v], rows_v, sem).wait()
    pltpu.sync_copy(rows_v, out_hbm.at[pl.ds(base, BPW)])


def kernel(all_answers, embedding_table):
    flat = _detile(embedding_table.T)
    return _gather_kernel(all_answers, flat.reshape(V, D))


# speedup vs baseline: 8.7065x; 1.3515x over previous
import functools

import jax
import jax.numpy as jnp
from jax import lax
from jax.experimental import pallas as pl
from jax.experimental.pallas import tpu as pltpu
from jax.experimental.pallas import tpu_sc as plsc

V = 1_000_000
D = 64
B = 16384
NC = 2
NS = 16
NW = NC * NS
BPW = B // NW
LANES = 128
NTC = -(-V // LANES)


C = 2048
NB = -(-V // C)


def _transpose_block(in_ref, out_ref):
    z = in_ref[...].T
    zp = jnp.concatenate([z, jnp.zeros_like(z)], 1)
    s = zp.reshape(C // 2, 2, LANES)
    y = s[:, 0, :] + jnp.roll(s[:, 1, :], D, axis=1)
    out_ref[...] = y.reshape(C * D)


_detile = pl.pallas_call(
    _transpose_block,
    grid=(NB,),
    in_specs=[pl.BlockSpec((D, C), lambda j: (0, j))],
    out_specs=pl.BlockSpec((C * D,), lambda j: (j,)),
    out_shape=jax.ShapeDtypeStruct((V * D,), jnp.float32),
    compiler_params=pltpu.CompilerParams(
        dimension_semantics=("parallel",),
    ),
)

_mesh = plsc.VectorSubcoreMesh(core_axis_name="c", subcore_axis_name="s")


@functools.partial(
    pl.kernel,
    mesh=_mesh,
    out_type=jax.ShapeDtypeStruct((B, D), jnp.float32),
    scratch_types=[
        pltpu.VMEM((BPW,), jnp.int32),
        pltpu.VMEM((BPW, D), jnp.float32),
        pltpu.SemaphoreType.DMA,
    ],
    compiler_params=pltpu.CompilerParams(use_tc_tiling_on_sc=False),
)
def _gather_kernel(idx_hbm, tbl_hbm, out_hbm, idx_v, rows_v, sem):
    wid = lax.axis_index("s") * NC + lax.axis_index("c")
    base = wid * BPW
    pltpu.sync_copy(idx_hbm.at[pl.ds(base, BPW)], idx_v)
    pltpu.async_copy(tbl_hbm.at[idx_v], rows_v, sem).wait()
    pltpu.sync_copy(rows_v, out_hbm.at[pl.ds(base, BPW)])


def kernel(all_answers, embedding_table):
    flat = _detile(embedding_table.T)
    return _gather_kernel(all_answers, flat.reshape(V, D))


# speedup vs baseline: 9.9862x; 1.1470x over previous
import functools

import jax
import jax.numpy as jnp
from jax import lax
from jax.experimental import pallas as pl
from jax.experimental.pallas import tpu as pltpu
from jax.experimental.pallas import tpu_sc as plsc

V = 1_000_000
D = 64
B = 16384
NC = 2
NS = 16
NW = NC * NS
BPW = B // NW
LANES = 128
NTC = -(-V // LANES)


C = 8192
NB = -(-V // C)


def _transpose_block(in_ref, out_ref):
    z = in_ref[...].T
    zp = jnp.concatenate([z, jnp.zeros_like(z)], 1)
    s = zp.reshape(C // 2, 2, LANES)
    y = s[:, 0, :] + jnp.roll(s[:, 1, :], D, axis=1)
    out_ref[...] = y.reshape(C * D)


_detile = pl.pallas_call(
    _transpose_block,
    grid=(NB,),
    in_specs=[pl.BlockSpec((D, C), lambda j: (0, j))],
    out_specs=pl.BlockSpec((C * D,), lambda j: (j,)),
    out_shape=jax.ShapeDtypeStruct((V * D,), jnp.float32),
    compiler_params=pltpu.CompilerParams(
        dimension_semantics=("parallel",),
    ),
)

_mesh = plsc.VectorSubcoreMesh(core_axis_name="c", subcore_axis_name="s")


@functools.partial(
    pl.kernel,
    mesh=_mesh,
    out_type=jax.ShapeDtypeStruct((B, D), jnp.float32),
    scratch_types=[
        pltpu.VMEM((BPW,), jnp.int32),
        pltpu.VMEM((BPW, D), jnp.float32),
        pltpu.SemaphoreType.DMA,
    ],
    compiler_params=pltpu.CompilerParams(use_tc_tiling_on_sc=False),
)
def _gather_kernel(idx_hbm, tbl_hbm, out_hbm, idx_v, rows_v, sem):
    wid = lax.axis_index("s") * NC + lax.axis_index("c")
    base = wid * BPW
    pltpu.sync_copy(idx_hbm.at[pl.ds(base, BPW)], idx_v)
    pltpu.async_copy(tbl_hbm.at[idx_v], rows_v, sem).wait()
    pltpu.sync_copy(rows_v, out_hbm.at[pl.ds(base, BPW)])


def kernel(all_answers, embedding_table):
    flat = _detile(embedding_table.T)
    return _gather_kernel(all_answers, flat.reshape(V, D))


# speedup vs baseline: 10.0307x; 1.0045x over previous
import functools

import jax
import jax.numpy as jnp
from jax import lax
from jax.experimental import pallas as pl
from jax.experimental.pallas import tpu as pltpu
from jax.experimental.pallas import tpu_sc as plsc

V = 1_000_000
D = 64
B = 16384
NC = 2
NS = 16
NW = NC * NS
BPW = B // NW
LANES = 128
NTC = -(-V // LANES)


C = 16384
NB = -(-V // C)


def _transpose_block(in_ref, out_ref):
    z = in_ref[...].T
    zp = jnp.concatenate([z, jnp.zeros_like(z)], 1)
    s = zp.reshape(C // 2, 2, LANES)
    y = s[:, 0, :] + jnp.roll(s[:, 1, :], D, axis=1)
    out_ref[...] = y.reshape(C * D)


_detile = pl.pallas_call(
    _transpose_block,
    grid=(NB,),
    in_specs=[pl.BlockSpec((D, C), lambda j: (0, j))],
    out_specs=pl.BlockSpec((C * D,), lambda j: (j,)),
    out_shape=jax.ShapeDtypeStruct((V * D,), jnp.float32),
    compiler_params=pltpu.CompilerParams(
        dimension_semantics=("parallel",),
    ),
)

_mesh = plsc.VectorSubcoreMesh(core_axis_name="c", subcore_axis_name="s")


@functools.partial(
    pl.kernel,
    mesh=_mesh,
    out_type=jax.ShapeDtypeStruct((B, D), jnp.float32),
    scratch_types=[
        pltpu.VMEM((BPW,), jnp.int32),
        pltpu.VMEM((BPW, D), jnp.float32),
        pltpu.SemaphoreType.DMA,
    ],
    compiler_params=pltpu.CompilerParams(use_tc_tiling_on_sc=False),
)
def _gather_kernel(idx_hbm, tbl_hbm, out_hbm, idx_v, rows_v, sem):
    wid = lax.axis_index("s") * NC + lax.axis_index("c")
    base = wid * BPW
    pltpu.sync_copy(idx_hbm.at[pl.ds(base, BPW)], idx_v)
    pltpu.async_copy(tbl_hbm.at[idx_v], rows_v, sem).wait()
    pltpu.sync_copy(rows_v, out_hbm.at[pl.ds(base, BPW)])


def kernel(all_answers, embedding_table):
    flat = _detile(embedding_table.T)
    return _gather_kernel(all_answers, flat.reshape(V, D))
